# baseline (speedup 1.0000x reference)
import functools

import jax
import jax.numpy as jnp
from jax import lax
from jax.experimental import pallas as pl
from jax.experimental.pallas import tpu as pltpu
from jax.experimental.pallas import tpu_sc as plsc

N = 10000
E = 320000
D = 128

NC = 2
NS = 16
CHUNK = 128
NPAD = 10112
EPAD = NC * NS * 80 * CHUNK
NCHUNKS = EPAD // (NC * NS * CHUNK)
RPT = NPAD // NS

_mesh = plsc.VectorSubcoreMesh(core_axis_name="c", subcore_axis_name="s")


@functools.partial(
    pl.kernel, mesh=_mesh,
    out_type=jax.ShapeDtypeStruct((NC, NPAD, D), jnp.float32),
    scratch_types=[
        pltpu.VMEM((NCHUNKS, CHUNK), jnp.int32),
        pltpu.VMEM((CHUNK, D), jnp.float32),
        pltpu.VMEM_SHARED((NPAD, D), jnp.float32),
    ],
)
def _sc_degree(dst_hbm, ones_hbm, z_hbm, out_hbm, didx_v, ones_v, acc_sh):
    cid = lax.axis_index("c")
    sid = lax.axis_index("s")
    pltpu.sync_copy(z_hbm, acc_sh.at[pl.ds(sid * RPT, RPT)])
    pltpu.sync_copy(dst_hbm.at[cid, sid], didx_v)
    pltpu.sync_copy(ones_hbm, ones_v)
    plsc.subcore_barrier()

    def body(j, c):
        pltpu.sync_copy(ones_v, acc_sh.at[didx_v.at[j]], add=True)
        return c

    lax.fori_loop(0, NCHUNKS, body, 0)
    plsc.subcore_barrier()
    pltpu.sync_copy(acc_sh.at[pl.ds(sid * RPT, RPT)],
                    out_hbm.at[cid, pl.ds(sid * RPT, RPT)])


NBUF = 3
NSLOT = NBUF * ((NCHUNKS + 2 + NBUF - 1) // NBUF)


@functools.partial(
    pl.kernel, mesh=_mesh,
    out_type=jax.ShapeDtypeStruct((NC, NPAD, D), jnp.float32),
    scratch_types=[
        pltpu.VMEM((NBUF, CHUNK), jnp.int32),
        pltpu.VMEM((NBUF, CHUNK), jnp.int32),
        pltpu.VMEM((NBUF, CHUNK, D), jnp.float32),
        pltpu.VMEM_SHARED((NPAD, D), jnp.float32),
    ] + [pltpu.SemaphoreType.DMA] * (2 * NBUF),
)
def _sc_aggregate(yw_hbm, src_hbm, dst_hbm, z_hbm, out_hbm,
                  sidx_v, didx_v, rows_v, acc_sh, *sems):
    gsems, ssems = sems[:NBUF], sems[NBUF:]
    cid = lax.axis_index("c")
    sid = lax.axis_index("s")
    pltpu.sync_copy(z_hbm, acc_sh.at[pl.ds(sid * RPT, RPT)])
    plsc.subcore_barrier()

    def _g(b):
        bb = jnp.int32(b)
        return pltpu.make_async_copy(
            yw_hbm.at[sidx_v.at[bb]], rows_v.at[bb], gsems[b])

    def _s(b):
        bb = jnp.int32(b)
        return pltpu.make_async_copy(
            rows_v.at[bb], acc_sh.at[didx_v.at[bb]], ssems[b])

    def _stage(j, b):
        bb = jnp.int32(b)
        pltpu.sync_copy(src_hbm.at[cid, sid, j], sidx_v.at[bb])
        pltpu.sync_copy(dst_hbm.at[cid, sid, j], didx_v.at[bb])
        _g(b).start()

    _stage(jnp.int32(0), 0)

    def outer(i, c):
        j0 = i * jnp.int32(NBUF)
        for p in range(NBUF):
            j = j0 + jnp.int32(p)
            b0 = p
            b1 = (p + 1) % NBUF
            pl.when(jnp.logical_and(j >= 2, j <= NCHUNKS + 1))(
                lambda: _s(b1).wait())
            pl.when(j + 1 <= NCHUNKS - 1)(lambda: _stage(j + 1, b1))

            def _work(b=b0):
                _g(b).wait()
                _s(b).start(add=True)

            pl.when(j <= NCHUNKS - 1)(_work)
        return c

    lax.fori_loop(jnp.int32(0), jnp.int32(NSLOT // NBUF), outer, 0)
    plsc.subcore_barrier()
    pltpu.sync_copy(acc_sh.at[pl.ds(sid * RPT, RPT)],
                    out_hbm.at[cid, pl.ds(sid * RPT, RPT)])


def _dinv_from(deg_ref):
    d = deg_ref[...]
    deg = d[0, :, 0:1] + d[1, :, 0:1] + 1.0
    return lax.rsqrt(deg)


def _tc_yw0_body(x_ref, w_ref, deg_ref, o_ref):
    dinv = _dinv_from(deg_ref)
    xw = jnp.dot(x_ref[...], w_ref[...], preferred_element_type=jnp.float32)
    o_ref[...] = xw * dinv


def _bn_relu(out, g, beta):
    mean = jnp.mean(out, axis=0, keepdims=True)
    var = jnp.mean((out - mean) ** 2, axis=0, keepdims=True)
    return jnp.maximum((out - mean) * lax.rsqrt(var + 1e-5) * g + beta, 0.0)


def _tc_mid_body(acc_ref, yw_ref, deg_ref, g_ref, beta_ref, b_ref, w_ref, o_ref):
    dinv = _dinv_from(deg_ref)
    acc = acc_ref[0, :N, :] + acc_ref[1, :N, :] + yw_ref[...]
    out = acc * dinv + b_ref[...]
    h = _bn_relu(out, g_ref[...], beta_ref[...])
    o_ref[...] = jnp.dot(h, w_ref[...], preferred_element_type=jnp.float32) * dinv


def _tc_final_body(acc_ref, yw_ref, deg_ref, g_ref, beta_ref, b_ref,
                   lw1_ref, lb1_ref, lw2_ref, lb2_ref, lw3_ref, lb3_ref, o_ref):
    dinv = _dinv_from(deg_ref)
    acc = acc_ref[0, :N, :] + acc_ref[1, :N, :] + yw_ref[...]
    out = acc * dinv + b_ref[...]
    h = _bn_relu(out, g_ref[...], beta_ref[...])
    m = jnp.maximum(
        jnp.dot(h, lw1_ref[...], preferred_element_type=jnp.float32) + lb1_ref[...],
        0.0)
    m = jnp.maximum(
        jnp.dot(m, lw2_ref[...], preferred_element_type=jnp.float32) + lb2_ref[...],
        0.0)
    o_ref[...] = (
        jnp.dot(m, lw3_ref[...], preferred_element_type=jnp.float32) + lb3_ref[...])


_f32 = jnp.float32


def kernel(x, edge_index, W0, b0, g0, beta0, W1, b1, g1, beta1,
           lw1, lb1, lw2, lb2, lw3, lb3):
    x = x.astype(_f32)
    src = edge_index[0].astype(jnp.int32)
    dst = edge_index[1].astype(jnp.int32)
    npad = EPAD - E
    src3 = jnp.concatenate([src, jnp.zeros((npad,), jnp.int32)]
                           ).reshape(NC, NS, NCHUNKS, CHUNK)
    dst3 = jnp.concatenate([dst, jnp.full((npad,), N, jnp.int32)]
                           ).reshape(NC, NS, NCHUNKS, CHUNK)
    zrows = jnp.zeros((RPT, D), _f32)
    ones_rows = jnp.ones((CHUNK, D), _f32)

    degp = _sc_degree(dst3, ones_rows, zrows)
    degc = degp[:, :N, 0:8]

    b0r, g0r, beta0r = (v.reshape(1, D).astype(_f32) for v in (b0, g0, beta0))
    b1r, g1r, beta1r = (v.reshape(1, D).astype(_f32) for v in (b1, g1, beta1))
    lb1r, lb2r, lb3r = (v.reshape(1, D).astype(_f32) for v in (lb1, lb2, lb3))

    yw0 = pl.pallas_call(
        _tc_yw0_body,
        out_shape=jax.ShapeDtypeStruct((N, D), _f32),
    )(x, W0.astype(_f32), degc)

    acc0 = _sc_aggregate(yw0, src3, dst3, zrows)

    yw1 = pl.pallas_call(
        _tc_mid_body,
        out_shape=jax.ShapeDtypeStruct((N, D), _f32),
    )(acc0, yw0, degc, g0r, beta0r, b0r, W1.astype(_f32))

    acc1 = _sc_aggregate(yw1, src3, dst3, zrows)

    out = pl.pallas_call(
        _tc_final_body,
        out_shape=jax.ShapeDtypeStruct((N, D), _f32),
    )(acc1, yw1, degc, g1r, beta1r, b1r,
      lw1.astype(_f32), lb1r, lw2.astype(_f32), lb2r, lw3.astype(_f32), lb3r)
    return out


# speedup vs baseline: 1.0610x; 1.0610x over previous
import functools

import jax
import jax.numpy as jnp
from jax import lax
from jax.experimental import pallas as pl
from jax.experimental.pallas import tpu as pltpu
from jax.experimental.pallas import tpu_sc as plsc

N = 10000
E = 320000
D = 128

NC = 2
NS = 16
CHUNK = 128
NPAD = 10112
EPAD = NC * NS * 80 * CHUNK
NCHUNKS = EPAD // (NC * NS * CHUNK)
RPT = NPAD // NS

_mesh = plsc.VectorSubcoreMesh(core_axis_name="c", subcore_axis_name="s")


@functools.partial(
    pl.kernel, mesh=_mesh,
    out_type=jax.ShapeDtypeStruct((NC, NPAD, D), jnp.float32),
    scratch_types=[
        pltpu.VMEM((NCHUNKS, CHUNK), jnp.int32),
        pltpu.VMEM((CHUNK, D), jnp.float32),
        pltpu.VMEM_SHARED((NPAD, D), jnp.float32),
    ],
)
def _sc_degree(dst_hbm, ones_hbm, z_hbm, out_hbm, didx_v, ones_v, acc_sh):
    cid = lax.axis_index("c")
    sid = lax.axis_index("s")
    pltpu.sync_copy(z_hbm, acc_sh.at[pl.ds(sid * RPT, RPT)])
    pltpu.sync_copy(dst_hbm.at[cid, sid], didx_v)
    pltpu.sync_copy(ones_hbm, ones_v)
    plsc.subcore_barrier()

    def body(j, c):
        pltpu.sync_copy(ones_v, acc_sh.at[didx_v.at[j]], add=True)
        return c

    lax.fori_loop(0, NCHUNKS, body, 0)
    plsc.subcore_barrier()
    pltpu.sync_copy(acc_sh.at[pl.ds(sid * RPT, RPT)],
                    out_hbm.at[cid, pl.ds(sid * RPT, RPT)])


NBUF = 3
NSLOT = NBUF * ((NCHUNKS + 2 + NBUF - 1) // NBUF)


@functools.partial(
    pl.kernel, mesh=_mesh,
    out_type=jax.ShapeDtypeStruct((NC, NPAD, D), jnp.float32),
    scratch_types=[
        pltpu.VMEM((NBUF, 2, CHUNK), jnp.int32),
        pltpu.VMEM((NBUF, CHUNK, D), jnp.float32),
        pltpu.VMEM_SHARED((NPAD, D), jnp.float32),
    ] + [pltpu.SemaphoreType.DMA] * (3 * NBUF),
)
def _sc_aggregate(yw_hbm, idx2_hbm, z_hbm, out_hbm,
                  idx_v, rows_v, acc_sh, *sems):
    gsems, ssems, isems = sems[:NBUF], sems[NBUF:2 * NBUF], sems[2 * NBUF:]
    cid = lax.axis_index("c")
    sid = lax.axis_index("s")
    pltpu.sync_copy(z_hbm, acc_sh.at[pl.ds(sid * RPT, RPT)])
    plsc.subcore_barrier()

    def _g(b):
        bb = jnp.int32(b)
        return pltpu.make_async_copy(
            yw_hbm.at[idx_v.at[bb, jnp.int32(0)]], rows_v.at[bb], gsems[b])

    def _s(b):
        bb = jnp.int32(b)
        return pltpu.make_async_copy(
            rows_v.at[bb], acc_sh.at[idx_v.at[bb, jnp.int32(1)]], ssems[b])

    def _i(j, b):
        bb = jnp.int32(b)
        return pltpu.make_async_copy(
            idx2_hbm.at[cid, sid, j], idx_v.at[bb], isems[b])

    pltpu.sync_copy(idx2_hbm.at[cid, sid, jnp.int32(0)], idx_v.at[jnp.int32(0)])
    _g(0).start()

    def outer(i, c):
        j0 = i * jnp.int32(NBUF)
        for p in range(NBUF):
            j = j0 + jnp.int32(p)
            b0 = p
            b1 = (p + 1) % NBUF
            pl.when(jnp.logical_and(j >= 2, j <= NCHUNKS + 1))(
                lambda: _s(b1).wait())
            pl.when(j + 1 <= NCHUNKS - 1)(
                lambda: _i(j + jnp.int32(1), b1).start())

            def _work(b=b0):
                _g(b).wait()
                _s(b).start(add=True)

            pl.when(j <= NCHUNKS - 1)(_work)

            def _launch(jj=j, b=b1):
                _i(jj + jnp.int32(1), b).wait()
                _g(b).start()

            pl.when(j + 1 <= NCHUNKS - 1)(_launch)
        return c

    lax.fori_loop(jnp.int32(0), jnp.int32(NSLOT // NBUF), outer, 0)
    plsc.subcore_barrier()
    pltpu.sync_copy(acc_sh.at[pl.ds(sid * RPT, RPT)],
                    out_hbm.at[cid, pl.ds(sid * RPT, RPT)])


def _dinv_from(deg_ref):
    d = deg_ref[...]
    deg = d[0, :, 0:1] + d[1, :, 0:1] + 1.0
    return lax.rsqrt(deg)


def _tc_yw0_body(x_ref, w_ref, deg_ref, o_ref):
    dinv = _dinv_from(deg_ref)
    xw = jnp.dot(x_ref[...], w_ref[...], preferred_element_type=jnp.float32)
    o_ref[...] = xw * dinv


def _bn_relu(out, g, beta):
    mean = jnp.mean(out, axis=0, keepdims=True)
    var = jnp.mean((out - mean) ** 2, axis=0, keepdims=True)
    return jnp.maximum((out - mean) * lax.rsqrt(var + 1e-5) * g + beta, 0.0)


def _tc_mid_body(acc_ref, yw_ref, deg_ref, g_ref, beta_ref, b_ref, w_ref, o_ref):
    dinv = _dinv_from(deg_ref)
    acc = acc_ref[0, :N, :] + acc_ref[1, :N, :] + yw_ref[...]
    out = acc * dinv + b_ref[...]
    h = _bn_relu(out, g_ref[...], beta_ref[...])
    o_ref[...] = jnp.dot(h, w_ref[...], preferred_element_type=jnp.float32) * dinv


def _tc_final_body(acc_ref, yw_ref, deg_ref, g_ref, beta_ref, b_ref,
                   lw1_ref, lb1_ref, lw2_ref, lb2_ref, lw3_ref, lb3_ref, o_ref):
    dinv = _dinv_from(deg_ref)
    acc = acc_ref[0, :N, :] + acc_ref[1, :N, :] + yw_ref[...]
    out = acc * dinv + b_ref[...]
    h = _bn_relu(out, g_ref[...], beta_ref[...])
    m = jnp.maximum(
        jnp.dot(h, lw1_ref[...], preferred_element_type=jnp.float32) + lb1_ref[...],
        0.0)
    m = jnp.maximum(
        jnp.dot(m, lw2_ref[...], preferred_element_type=jnp.float32) + lb2_ref[...],
        0.0)
    o_ref[...] = (
        jnp.dot(m, lw3_ref[...], preferred_element_type=jnp.float32) + lb3_ref[...])


_f32 = jnp.float32


def kernel(x, edge_index, W0, b0, g0, beta0, W1, b1, g1, beta1,
           lw1, lb1, lw2, lb2, lw3, lb3):
    x = x.astype(_f32)
    src = edge_index[0].astype(jnp.int32)
    dst = edge_index[1].astype(jnp.int32)
    npad = EPAD - E
    src3 = jnp.concatenate([src, jnp.zeros((npad,), jnp.int32)]
                           ).reshape(NC, NS, NCHUNKS, CHUNK)
    dst3 = jnp.concatenate([dst, jnp.full((npad,), N, jnp.int32)]
                           ).reshape(NC, NS, NCHUNKS, CHUNK)
    idx2 = jnp.stack([src3, dst3], axis=3)
    zrows = jnp.zeros((RPT, D), _f32)
    ones_rows = jnp.ones((CHUNK, D), _f32)

    degp = _sc_degree(dst3, ones_rows, zrows)
    degc = degp[:, :N, 0:8]

    b0r, g0r, beta0r = (v.reshape(1, D).astype(_f32) for v in (b0, g0, beta0))
    b1r, g1r, beta1r = (v.reshape(1, D).astype(_f32) for v in (b1, g1, beta1))
    lb1r, lb2r, lb3r = (v.reshape(1, D).astype(_f32) for v in (lb1, lb2, lb3))

    yw0 = pl.pallas_call(
        _tc_yw0_body,
        out_shape=jax.ShapeDtypeStruct((N, D), _f32),
    )(x, W0.astype(_f32), degc)

    acc0 = _sc_aggregate(yw0, idx2, zrows)

    yw1 = pl.pallas_call(
        _tc_mid_body,
        out_shape=jax.ShapeDtypeStruct((N, D), _f32),
    )(acc0, yw0, degc, g0r, beta0r, b0r, W1.astype(_f32))

    acc1 = _sc_aggregate(yw1, idx2, zrows)

    out = pl.pallas_call(
        _tc_final_body,
        out_shape=jax.ShapeDtypeStruct((N, D), _f32),
    )(acc1, yw1, degc, g1r, beta1r, b1r,
      lw1.astype(_f32), lb1r, lw2.astype(_f32), lb2r, lw3.astype(_f32), lb3r)
    return out


# speedup vs baseline: 1.0792x; 1.0171x over previous
import functools

import jax
import jax.numpy as jnp
from jax import lax
from jax.experimental import pallas as pl
from jax.experimental.pallas import tpu as pltpu
from jax.experimental.pallas import tpu_sc as plsc

N = 10000
E = 320000
D = 128

NC = 2
NS = 16
CHUNK = 128
NPAD = 10112
EPAD = NC * NS * 80 * CHUNK
NCHUNKS = EPAD // (NC * NS * CHUNK)
RPT = NPAD // NS

_mesh = plsc.VectorSubcoreMesh(core_axis_name="c", subcore_axis_name="s")


@functools.partial(
    pl.kernel, mesh=_mesh,
    out_type=jax.ShapeDtypeStruct((NC, NPAD, D), jnp.float32),
    scratch_types=[
        pltpu.VMEM((NCHUNKS, CHUNK), jnp.int32),
        pltpu.VMEM((CHUNK, D), jnp.float32),
        pltpu.VMEM_SHARED((NPAD, D), jnp.float32),
    ],
)
def _sc_degree(dst_hbm, ones_hbm, z_hbm, out_hbm, didx_v, ones_v, acc_sh):
    cid = lax.axis_index("c")
    sid = lax.axis_index("s")
    pltpu.sync_copy(z_hbm, acc_sh.at[pl.ds(sid * RPT, RPT)])
    pltpu.sync_copy(dst_hbm.at[cid, sid], didx_v)
    pltpu.sync_copy(ones_hbm, ones_v)
    plsc.subcore_barrier()

    def body(j, c):
        pltpu.sync_copy(ones_v, acc_sh.at[didx_v.at[j]], add=True)
        return c

    lax.fori_loop(0, NCHUNKS, body, 0)
    plsc.subcore_barrier()
    pltpu.sync_copy(acc_sh.at[pl.ds(sid * RPT, RPT)],
                    out_hbm.at[cid, pl.ds(sid * RPT, RPT)])


GBUF = 2
IBUF = 4


@functools.partial(
    pl.kernel, mesh=_mesh,
    out_type=jax.ShapeDtypeStruct((NC, NPAD, D), jnp.float32),
    scratch_types=[
        pltpu.VMEM((IBUF, 2, CHUNK), jnp.int32),
        pltpu.VMEM((GBUF, CHUNK, D), jnp.float32),
        pltpu.VMEM_SHARED((NPAD, D), jnp.float32),
    ] + [pltpu.SemaphoreType.DMA] * (GBUF + IBUF),
)
def _sc_aggregate(yw_hbm, idx2_hbm, z_hbm, out_hbm,
                  idx_v, rows_v, acc_sh, *sems):
    gsems, isems = sems[:GBUF], sems[GBUF:]
    cid = lax.axis_index("c")
    sid = lax.axis_index("s")
    pltpu.sync_copy(z_hbm, acc_sh.at[pl.ds(sid * RPT, RPT)])
    plsc.subcore_barrier()

    def _g(b, q):
        return pltpu.make_async_copy(
            yw_hbm.at[idx_v.at[jnp.int32(q), jnp.int32(0)]],
            rows_v.at[jnp.int32(b)], gsems[b])

    def _i(j, q):
        return pltpu.make_async_copy(
            idx2_hbm.at[cid, sid, j], idx_v.at[jnp.int32(q)], isems[q])

    for q in (0, 1):
        pltpu.sync_copy(idx2_hbm.at[cid, sid, jnp.int32(q)],
                        idx_v.at[jnp.int32(q)])
    for q in (2, 3):
        _i(jnp.int32(q), q).start()
    _g(0, 0).start()
    _g(1, 1).start()

    def outer(i, c):
        j0 = i * jnp.int32(IBUF)
        for p in range(IBUF):
            j = j0 + jnp.int32(p)
            b = p % GBUF
            q = p
            q2 = (p + 2) % IBUF
            _g(b, q).wait()
            pltpu.sync_copy(rows_v.at[jnp.int32(b)],
                            acc_sh.at[idx_v.at[jnp.int32(q), jnp.int32(1)]],
                            add=True)
            pl.when(j + IBUF <= NCHUNKS - 1)(
                lambda: _i(j + jnp.int32(IBUF), q).start())

            def _launch(jj=j, b=b, q2=q2):
                _i(jj + jnp.int32(2), q2).wait()
                _g(b, q2).start()

            pl.when(j + 2 <= NCHUNKS - 1)(_launch)
        return c

    lax.fori_loop(jnp.int32(0), jnp.int32(NCHUNKS // IBUF), outer, 0)
    plsc.subcore_barrier()
    pltpu.sync_copy(acc_sh.at[pl.ds(sid * RPT, RPT)],
                    out_hbm.at[cid, pl.ds(sid * RPT, RPT)])


def _dinv_from(deg_ref):
    d = deg_ref[...]
    deg = d[0, :, 0:1] + d[1, :, 0:1] + 1.0
    return lax.rsqrt(deg)


def _tc_yw0_body(x_ref, w_ref, deg_ref, o_ref):
    dinv = _dinv_from(deg_ref)
    xw = jnp.dot(x_ref[...], w_ref[...], preferred_element_type=jnp.float32)
    o_ref[...] = xw * dinv


def _bn_relu(out, g, beta):
    mean = jnp.mean(out, axis=0, keepdims=True)
    var = jnp.mean((out - mean) ** 2, axis=0, keepdims=True)
    return jnp.maximum((out - mean) * lax.rsqrt(var + 1e-5) * g + beta, 0.0)


def _tc_mid_body(acc_ref, yw_ref, deg_ref, g_ref, beta_ref, b_ref, w_ref, o_ref):
    dinv = _dinv_from(deg_ref)
    acc = acc_ref[0, :N, :] + acc_ref[1, :N, :] + yw_ref[...]
    out = acc * dinv + b_ref[...]
    h = _bn_relu(out, g_ref[...], beta_ref[...])
    o_ref[...] = jnp.dot(h, w_ref[...], preferred_element_type=jnp.float32) * dinv


def _tc_final_body(acc_ref, yw_ref, deg_ref, g_ref, beta_ref, b_ref,
                   lw1_ref, lb1_ref, lw2_ref, lb2_ref, lw3_ref, lb3_ref, o_ref):
    dinv = _dinv_from(deg_ref)
    acc = acc_ref[0, :N, :] + acc_ref[1, :N, :] + yw_ref[...]
    out = acc * dinv + b_ref[...]
    h = _bn_relu(out, g_ref[...], beta_ref[...])
    m = jnp.maximum(
        jnp.dot(h, lw1_ref[...], preferred_element_type=jnp.float32) + lb1_ref[...],
        0.0)
    m = jnp.maximum(
        jnp.dot(m, lw2_ref[...], preferred_element_type=jnp.float32) + lb2_ref[...],
        0.0)
    o_ref[...] = (
        jnp.dot(m, lw3_ref[...], preferred_element_type=jnp.float32) + lb3_ref[...])


_f32 = jnp.float32


def kernel(x, edge_index, W0, b0, g0, beta0, W1, b1, g1, beta1,
           lw1, lb1, lw2, lb2, lw3, lb3):
    x = x.astype(_f32)
    src = edge_index[0].astype(jnp.int32)
    dst = edge_index[1].astype(jnp.int32)
    npad = EPAD - E
    src3 = jnp.concatenate([src, jnp.zeros((npad,), jnp.int32)]
                           ).reshape(NC, NS, NCHUNKS, CHUNK)
    dst3 = jnp.concatenate([dst, jnp.full((npad,), N, jnp.int32)]
                           ).reshape(NC, NS, NCHUNKS, CHUNK)
    idx2 = jnp.stack([src3, dst3], axis=3)
    zrows = jnp.zeros((RPT, D), _f32)
    ones_rows = jnp.ones((CHUNK, D), _f32)

    degp = _sc_degree(dst3, ones_rows, zrows)
    degc = degp[:, :N, 0:8]

    b0r, g0r, beta0r = (v.reshape(1, D).astype(_f32) for v in (b0, g0, beta0))
    b1r, g1r, beta1r = (v.reshape(1, D).astype(_f32) for v in (b1, g1, beta1))
    lb1r, lb2r, lb3r = (v.reshape(1, D).astype(_f32) for v in (lb1, lb2, lb3))

    yw0 = pl.pallas_call(
        _tc_yw0_body,
        out_shape=jax.ShapeDtypeStruct((N, D), _f32),
    )(x, W0.astype(_f32), degc)

    acc0 = _sc_aggregate(yw0, idx2, zrows)

    yw1 = pl.pallas_call(
        _tc_mid_body,
        out_shape=jax.ShapeDtypeStruct((N, D), _f32),
    )(acc0, yw0, degc, g0r, beta0r, b0r, W1.astype(_f32))

    acc1 = _sc_aggregate(yw1, idx2, zrows)

    out = pl.pallas_call(
        _tc_final_body,
        out_shape=jax.ShapeDtypeStruct((N, D), _f32),
    )(acc1, yw1, degc, g1r, beta1r, b1r,
      lw1.astype(_f32), lb1r, lw2.astype(_f32), lb2r, lw3.astype(_f32), lb3r)
    return out


# speedup vs baseline: 1.1045x; 1.0235x over previous
import functools

import jax
import jax.numpy as jnp
from jax import lax
from jax.experimental import pallas as pl
from jax.experimental.pallas import tpu as pltpu
from jax.experimental.pallas import tpu_sc as plsc

N = 10000
E = 320000
D = 128

NC = 2
NS = 16
CHUNK = 128
NPAD = 10112
EPAD = NC * NS * 80 * CHUNK
NCHUNKS = EPAD // (NC * NS * CHUNK)
RPT = NPAD // NS

_mesh = plsc.VectorSubcoreMesh(core_axis_name="c", subcore_axis_name="s")


@functools.partial(
    pl.kernel, mesh=_mesh,
    out_type=jax.ShapeDtypeStruct((NC, NPAD, D), jnp.float32),
    scratch_types=[
        pltpu.VMEM((NCHUNKS, CHUNK), jnp.int32),
        pltpu.VMEM((CHUNK, D), jnp.float32),
        pltpu.VMEM_SHARED((NPAD, D), jnp.float32),
    ],
)
def _sc_degree(dst_hbm, ones_hbm, z_hbm, out_hbm, didx_v, ones_v, acc_sh):
    cid = lax.axis_index("c")
    sid = lax.axis_index("s")
    pltpu.sync_copy(z_hbm, acc_sh.at[pl.ds(sid * RPT, RPT)])
    pltpu.sync_copy(dst_hbm.at[cid, sid], didx_v)
    pltpu.sync_copy(ones_hbm, ones_v)
    plsc.subcore_barrier()

    def body(j, c):
        pltpu.sync_copy(ones_v, acc_sh.at[didx_v.at[j]], add=True)
        return c

    lax.fori_loop(0, NCHUNKS, body, 0)
    plsc.subcore_barrier()
    pltpu.sync_copy(acc_sh.at[pl.ds(sid * RPT, RPT)],
                    out_hbm.at[cid, pl.ds(sid * RPT, RPT)])


GBUF = 3
IBUF = 6
NPA = 10008
RPA = 632
TAILA = NPA - 15 * RPA


@functools.partial(
    pl.kernel, mesh=_mesh,
    out_type=jax.ShapeDtypeStruct((NC, NPA, D), jnp.float32),
    scratch_types=[
        pltpu.VMEM((IBUF, 2, CHUNK), jnp.int32),
        pltpu.VMEM((GBUF, CHUNK, D), jnp.float32),
        pltpu.VMEM_SHARED((NPA, D), jnp.float32),
    ] + [pltpu.SemaphoreType.DMA] * (GBUF + IBUF),
)
def _sc_aggregate(yw_hbm, idx2_hbm, z_hbm, out_hbm,
                  idx_v, rows_v, acc_sh, *sems):
    gsems, isems = sems[:GBUF], sems[GBUF:]
    cid = lax.axis_index("c")
    sid = lax.axis_index("s")

    @pl.when(sid < NS - 1)
    def _():
        pltpu.sync_copy(z_hbm, acc_sh.at[pl.ds(sid * RPA, RPA)])

    @pl.when(sid == NS - 1)
    def _():
        pltpu.sync_copy(z_hbm.at[pl.ds(0, TAILA)],
                        acc_sh.at[pl.ds(15 * RPA, TAILA)])

    plsc.subcore_barrier()

    def _g(b, q):
        return pltpu.make_async_copy(
            yw_hbm.at[idx_v.at[jnp.int32(q), jnp.int32(0)]],
            rows_v.at[jnp.int32(b)], gsems[b])

    def _i(j, q):
        return pltpu.make_async_copy(
            idx2_hbm.at[cid, sid, j], idx_v.at[jnp.int32(q)], isems[q])

    for q in (0, 1, 2):
        pltpu.sync_copy(idx2_hbm.at[cid, sid, jnp.int32(q)],
                        idx_v.at[jnp.int32(q)])
    for q in (3, 4, 5):
        _i(jnp.int32(q), q).start()
    for b in (0, 1, 2):
        _g(b, b).start()

    def outer(i, c):
        j0 = i * jnp.int32(IBUF)
        for p in range(IBUF):
            j = j0 + jnp.int32(p)
            b = p % GBUF
            q = p
            q3 = (p + 3) % IBUF

            def _work(b=b, q=q):
                _g(b, q).wait()
                pltpu.sync_copy(
                    rows_v.at[jnp.int32(b)],
                    acc_sh.at[idx_v.at[jnp.int32(q), jnp.int32(1)]],
                    add=True)

            pl.when(j <= NCHUNKS - 1)(_work)
            pl.when(j + IBUF <= NCHUNKS - 1)(
                lambda: _i(j + jnp.int32(IBUF), q).start())

            def _launch(jj=j, b=b, q3=q3):
                _i(jj + jnp.int32(3), q3).wait()
                _g(b, q3).start()

            pl.when(j + 3 <= NCHUNKS - 1)(_launch)
        return c

    lax.fori_loop(jnp.int32(0), jnp.int32((NCHUNKS + IBUF) // IBUF), outer, 0)
    plsc.subcore_barrier()

    @pl.when(sid < NS - 1)
    def _():
        pltpu.sync_copy(acc_sh.at[pl.ds(sid * RPA, RPA)],
                        out_hbm.at[cid, pl.ds(sid * RPA, RPA)])

    @pl.when(sid == NS - 1)
    def _():
        pltpu.sync_copy(acc_sh.at[pl.ds(15 * RPA, TAILA)],
                        out_hbm.at[cid, pl.ds(15 * RPA, TAILA)])


def _dinv_from(deg_ref):
    d = deg_ref[...]
    deg = d[0, :, 0:1] + d[1, :, 0:1] + 1.0
    return lax.rsqrt(deg)


def _tc_yw0_body(x_ref, w_ref, deg_ref, o_ref):
    dinv = _dinv_from(deg_ref)
    xw = jnp.dot(x_ref[...], w_ref[...], preferred_element_type=jnp.float32)
    o_ref[...] = xw * dinv


def _bn_relu(out, g, beta):
    mean = jnp.mean(out, axis=0, keepdims=True)
    var = jnp.mean((out - mean) ** 2, axis=0, keepdims=True)
    return jnp.maximum((out - mean) * lax.rsqrt(var + 1e-5) * g + beta, 0.0)


def _tc_mid_body(acc_ref, yw_ref, deg_ref, g_ref, beta_ref, b_ref, w_ref, o_ref):
    dinv = _dinv_from(deg_ref)
    acc = acc_ref[0, :N, :] + acc_ref[1, :N, :] + yw_ref[...]
    out = acc * dinv + b_ref[...]
    h = _bn_relu(out, g_ref[...], beta_ref[...])
    o_ref[...] = jnp.dot(h, w_ref[...], preferred_element_type=jnp.float32) * dinv


def _tc_final_body(acc_ref, yw_ref, deg_ref, g_ref, beta_ref, b_ref,
                   lw1_ref, lb1_ref, lw2_ref, lb2_ref, lw3_ref, lb3_ref, o_ref):
    dinv = _dinv_from(deg_ref)
    acc = acc_ref[0, :N, :] + acc_ref[1, :N, :] + yw_ref[...]
    out = acc * dinv + b_ref[...]
    h = _bn_relu(out, g_ref[...], beta_ref[...])
    m = jnp.maximum(
        jnp.dot(h, lw1_ref[...], preferred_element_type=jnp.float32) + lb1_ref[...],
        0.0)
    m = jnp.maximum(
        jnp.dot(m, lw2_ref[...], preferred_element_type=jnp.float32) + lb2_ref[...],
        0.0)
    o_ref[...] = (
        jnp.dot(m, lw3_ref[...], preferred_element_type=jnp.float32) + lb3_ref[...])


_f32 = jnp.float32


def kernel(x, edge_index, W0, b0, g0, beta0, W1, b1, g1, beta1,
           lw1, lb1, lw2, lb2, lw3, lb3):
    x = x.astype(_f32)
    src = edge_index[0].astype(jnp.int32)
    dst = edge_index[1].astype(jnp.int32)
    npad = EPAD - E
    src3 = jnp.concatenate([src, jnp.zeros((npad,), jnp.int32)]
                           ).reshape(NC, NS, NCHUNKS, CHUNK)
    dst3 = jnp.concatenate([dst, jnp.full((npad,), N, jnp.int32)]
                           ).reshape(NC, NS, NCHUNKS, CHUNK)
    idx2 = jnp.stack([src3, dst3], axis=3)
    zrows = jnp.zeros((RPT, D), _f32)
    ones_rows = jnp.ones((CHUNK, D), _f32)

    degp = _sc_degree(dst3, ones_rows, zrows)
    degc = degp[:, :N, 0:8]

    b0r, g0r, beta0r = (v.reshape(1, D).astype(_f32) for v in (b0, g0, beta0))
    b1r, g1r, beta1r = (v.reshape(1, D).astype(_f32) for v in (b1, g1, beta1))
    lb1r, lb2r, lb3r = (v.reshape(1, D).astype(_f32) for v in (lb1, lb2, lb3))

    yw0 = pl.pallas_call(
        _tc_yw0_body,
        out_shape=jax.ShapeDtypeStruct((N, D), _f32),
    )(x, W0.astype(_f32), degc)

    acc0 = _sc_aggregate(yw0, idx2, zrows)

    yw1 = pl.pallas_call(
        _tc_mid_body,
        out_shape=jax.ShapeDtypeStruct((N, D), _f32),
    )(acc0, yw0, degc, g0r, beta0r, b0r, W1.astype(_f32))

    acc1 = _sc_aggregate(yw1, idx2, zrows)

    out = pl.pallas_call(
        _tc_final_body,
        out_shape=jax.ShapeDtypeStruct((N, D), _f32),
    )(acc1, yw1, degc, g1r, beta1r, b1r,
      lw1.astype(_f32), lb1r, lw2.astype(_f32), lb2r, lw3.astype(_f32), lb3r)
    return out


# speedup vs baseline: 1.8889x; 1.7102x over previous
import functools

import jax
import jax.numpy as jnp
from jax import lax
from jax.experimental import pallas as pl
from jax.experimental.pallas import tpu as pltpu
from jax.experimental.pallas import tpu_sc as plsc

N = 10000
E = 320000
D = 128

NC = 2
NS = 16
CHUNK = 128
NPAD = 10112
EPAD = NC * NS * 80 * CHUNK
NCHUNKS = EPAD // (NC * NS * CHUNK)
RPT = NPAD // NS

_mesh = plsc.VectorSubcoreMesh(core_axis_name="c", subcore_axis_name="s")


@functools.partial(
    pl.kernel, mesh=_mesh,
    out_type=jax.ShapeDtypeStruct((NC, NPAD, D), jnp.float32),
    scratch_types=[
        pltpu.VMEM((NCHUNKS, CHUNK), jnp.int32),
        pltpu.VMEM((CHUNK, D), jnp.float32),
        pltpu.VMEM_SHARED((NPAD, D), jnp.float32),
    ],
)
def _sc_degree(dst_hbm, ones_hbm, z_hbm, out_hbm, didx_v, ones_v, acc_sh):
    cid = lax.axis_index("c")
    sid = lax.axis_index("s")
    pltpu.sync_copy(z_hbm, acc_sh.at[pl.ds(sid * RPT, RPT)])
    pltpu.sync_copy(dst_hbm.at[cid, sid], didx_v)
    pltpu.sync_copy(ones_hbm, ones_v)
    plsc.subcore_barrier()

    def body(j, c):
        pltpu.sync_copy(ones_v, acc_sh.at[didx_v.at[j]], add=True)
        return c

    lax.fori_loop(0, NCHUNKS, body, 0)
    plsc.subcore_barrier()
    pltpu.sync_copy(acc_sh.at[pl.ds(sid * RPT, RPT)],
                    out_hbm.at[cid, pl.ds(sid * RPT, RPT)])


GBUF = 3
IBUF = 6
NPA = 10008
RPA = 632
TAILA = NPA - 15 * RPA
CH0 = 27
CH1 = 130
CHMAX = max(CH0, CH1)
E0 = NS * CHUNK * CH0


@functools.partial(
    pl.kernel, mesh=_mesh,
    out_type=jax.ShapeDtypeStruct((NC, NPA, D), jnp.float32),
    scratch_types=[
        pltpu.VMEM((IBUF, 2, CHUNK), jnp.int32),
        pltpu.VMEM((GBUF, CHUNK, D), jnp.float32),
        pltpu.VMEM_SHARED((NPA, D), jnp.float32),
    ] + [pltpu.SemaphoreType.DMA] * (GBUF + IBUF),
)
def _sc_aggregate(yw_hbm, idx2_hbm, z_hbm, out_hbm,
                  idx_v, rows_v, acc_sh, *sems):
    gsems, isems = sems[:GBUF], sems[GBUF:]
    cid = lax.axis_index("c")
    sid = lax.axis_index("s")
    L = jnp.where(cid == 0, jnp.int32(CH0), jnp.int32(CH1))

    @pl.when(sid < NS - 1)
    def _():
        pltpu.sync_copy(z_hbm, acc_sh.at[pl.ds(sid * RPA, RPA)])

    @pl.when(sid == NS - 1)
    def _():
        pltpu.sync_copy(z_hbm.at[pl.ds(0, TAILA)],
                        acc_sh.at[pl.ds(15 * RPA, TAILA)])

    plsc.subcore_barrier()

    def _g(b, q):
        return pltpu.make_async_copy(
            yw_hbm.at[idx_v.at[jnp.int32(q), jnp.int32(0)]],
            rows_v.at[jnp.int32(b)], gsems[b])

    def _i(j, q):
        return pltpu.make_async_copy(
            idx2_hbm.at[cid, sid, j], idx_v.at[jnp.int32(q)], isems[q])

    for q in (0, 1, 2):
        pltpu.sync_copy(idx2_hbm.at[cid, sid, jnp.int32(q)],
                        idx_v.at[jnp.int32(q)])
    for q in (3, 4, 5):
        _i(jnp.int32(q), q).start()
    for b in (0, 1, 2):
        _g(b, b).start()

    def outer(i, c):
        j0 = i * jnp.int32(IBUF)
        for p in range(IBUF):
            j = j0 + jnp.int32(p)
            b = p % GBUF
            q = p
            q3 = (p + 3) % IBUF

            def _work(b=b, q=q):
                _g(b, q).wait()
                pltpu.sync_copy(
                    rows_v.at[jnp.int32(b)],
                    acc_sh.at[idx_v.at[jnp.int32(q), jnp.int32(1)]],
                    add=True)

            pl.when(j <= L - 1)(_work)
            pl.when(j + IBUF <= L - 1)(
                lambda: _i(j + jnp.int32(IBUF), q).start())

            def _launch(jj=j, b=b, q3=q3):
                _i(jj + jnp.int32(3), q3).wait()
                _g(b, q3).start()

            pl.when(j + 3 <= L - 1)(_launch)
        return c

    lax.fori_loop(jnp.int32(0), jnp.int32((CHMAX + IBUF) // IBUF), outer, 0)
    plsc.subcore_barrier()

    @pl.when(sid < NS - 1)
    def _():
        pltpu.sync_copy(acc_sh.at[pl.ds(sid * RPA, RPA)],
                        out_hbm.at[cid, pl.ds(sid * RPA, RPA)])

    @pl.when(sid == NS - 1)
    def _():
        pltpu.sync_copy(acc_sh.at[pl.ds(15 * RPA, TAILA)],
                        out_hbm.at[cid, pl.ds(15 * RPA, TAILA)])


def _dinv_from(deg_ref):
    d = deg_ref[...]
    deg = d[0, :, 0:1] + d[1, :, 0:1] + 1.0
    return lax.rsqrt(deg)


def _tc_yw0_body(x_ref, w_ref, deg_ref, o_ref):
    dinv = _dinv_from(deg_ref)
    xw = jnp.dot(x_ref[...], w_ref[...], preferred_element_type=jnp.float32)
    o_ref[...] = xw * dinv


def _bn_relu(out, g, beta):
    mean = jnp.mean(out, axis=0, keepdims=True)
    var = jnp.mean((out - mean) ** 2, axis=0, keepdims=True)
    return jnp.maximum((out - mean) * lax.rsqrt(var + 1e-5) * g + beta, 0.0)


def _tc_mid_body(acc_ref, yw_ref, deg_ref, g_ref, beta_ref, b_ref, w_ref, o_ref):
    dinv = _dinv_from(deg_ref)
    acc = acc_ref[0, :N, :] + acc_ref[1, :N, :] + yw_ref[...]
    out = acc * dinv + b_ref[...]
    h = _bn_relu(out, g_ref[...], beta_ref[...])
    o_ref[...] = jnp.dot(h, w_ref[...], preferred_element_type=jnp.float32) * dinv


def _tc_final_body(acc_ref, yw_ref, deg_ref, g_ref, beta_ref, b_ref,
                   lw1_ref, lb1_ref, lw2_ref, lb2_ref, lw3_ref, lb3_ref, o_ref):
    dinv = _dinv_from(deg_ref)
    acc = acc_ref[0, :N, :] + acc_ref[1, :N, :] + yw_ref[...]
    out = acc * dinv + b_ref[...]
    h = _bn_relu(out, g_ref[...], beta_ref[...])
    m = jnp.maximum(
        jnp.dot(h, lw1_ref[...], preferred_element_type=jnp.float32) + lb1_ref[...],
        0.0)
    m = jnp.maximum(
        jnp.dot(m, lw2_ref[...], preferred_element_type=jnp.float32) + lb2_ref[...],
        0.0)
    o_ref[...] = (
        jnp.dot(m, lw3_ref[...], preferred_element_type=jnp.float32) + lb3_ref[...])


_f32 = jnp.float32


def kernel(x, edge_index, W0, b0, g0, beta0, W1, b1, g1, beta1,
           lw1, lb1, lw2, lb2, lw3, lb3):
    x = x.astype(_f32)
    src = edge_index[0].astype(jnp.int32)
    dst = edge_index[1].astype(jnp.int32)
    npad = EPAD - E
    src3 = jnp.concatenate([src, jnp.zeros((npad,), jnp.int32)]
                           ).reshape(NC, NS, NCHUNKS, CHUNK)
    dst3 = jnp.concatenate([dst, jnp.full((npad,), N, jnp.int32)]
                           ).reshape(NC, NS, NCHUNKS, CHUNK)
    pad1 = NS * CHUNK * CH1 - (E - E0)
    s0 = jnp.pad(src[:E0].reshape(NS, CH0, CHUNK),
                 ((0, 0), (0, CHMAX - CH0), (0, 0)))
    d0 = jnp.pad(dst[:E0].reshape(NS, CH0, CHUNK),
                 ((0, 0), (0, CHMAX - CH0), (0, 0)), constant_values=N)
    s1 = jnp.concatenate([src[E0:], jnp.zeros((pad1,), jnp.int32)]
                         ).reshape(NS, CH1, CHUNK)
    d1 = jnp.concatenate([dst[E0:], jnp.full((pad1,), N, jnp.int32)]
                         ).reshape(NS, CH1, CHUNK)
    s1 = jnp.pad(s1, ((0, 0), (0, CHMAX - CH1), (0, 0)))
    d1 = jnp.pad(d1, ((0, 0), (0, CHMAX - CH1), (0, 0)), constant_values=N)
    idx2 = jnp.stack([jnp.stack([s0, d0], axis=2),
                      jnp.stack([s1, d1], axis=2)])
    zrows = jnp.zeros((RPT, D), _f32)
    ones_rows = jnp.ones((CHUNK, D), _f32)

    degp = _sc_degree(dst3, ones_rows, zrows)
    degc = degp[:, :N, 0:8]

    b0r, g0r, beta0r = (v.reshape(1, D).astype(_f32) for v in (b0, g0, beta0))
    b1r, g1r, beta1r = (v.reshape(1, D).astype(_f32) for v in (b1, g1, beta1))
    lb1r, lb2r, lb3r = (v.reshape(1, D).astype(_f32) for v in (lb1, lb2, lb3))

    yw0 = pl.pallas_call(
        _tc_yw0_body,
        out_shape=jax.ShapeDtypeStruct((N, D), _f32),
    )(x, W0.astype(_f32), degc)

    acc0 = _sc_aggregate(yw0, idx2, zrows)

    yw1 = pl.pallas_call(
        _tc_mid_body,
        out_shape=jax.ShapeDtypeStruct((N, D), _f32),
    )(acc0, yw0, degc, g0r, beta0r, b0r, W1.astype(_f32))

    acc1 = _sc_aggregate(yw1, idx2, zrows)

    out = pl.pallas_call(
        _tc_final_body,
        out_shape=jax.ShapeDtypeStruct((N, D), _f32),
    )(acc1, yw1, degc, g1r, beta1r, b1r,
      lw1.astype(_f32), lb1r, lw2.astype(_f32), lb2r, lw3.astype(_f32), lb3r)
    return out
